# baseline (device time: 85417 ns/iter reference)
import jax
import jax.numpy as jnp
from jax import lax
from jax.experimental import pallas as pl
from jax.experimental.pallas import tpu as pltpu

N_DEV = 4
M_TOT = 4096
D = 1024
M_PER = M_TOT // N_DEV


def kernel(partial, gamma):
    partial = partial.reshape(M_TOT, D)
    gamma = gamma.reshape(1, D)

    def body(p_ref, g_ref, out_ref, comm_ref, send_sems, recv_sems):
        my_x = lax.axis_index("x")
        j = lax.axis_index("y")
        my_z = lax.axis_index("z")
        left = (j - 1) % N_DEV
        right = (j + 1) % N_DEV

        barrier_sem = pltpu.get_barrier_semaphore()
        for nbr in (left, right):
            pl.semaphore_signal(
                barrier_sem, inc=1,
                device_id=(my_x, nbr, my_z),
                device_id_type=pl.DeviceIdType.MESH,
            )
        pl.semaphore_wait(barrier_sem, 2)

        start = ((j - 1) % N_DEV) * M_PER
        comm_ref[0] = p_ref[pl.ds(start, M_PER), :].astype(jnp.bfloat16)

        for h in range(N_DEV - 1):
            rdma = pltpu.make_async_remote_copy(
                src_ref=comm_ref.at[h],
                dst_ref=comm_ref.at[h + 1],
                send_sem=send_sems.at[h],
                recv_sem=recv_sems.at[h],
                device_id=(my_x, right, my_z),
                device_id_type=pl.DeviceIdType.MESH,
            )
            rdma.start()
            rdma.wait()

            c = ((j - 2 - h) % N_DEV) * M_PER
            acc = comm_ref[h + 1].astype(jnp.float32) + p_ref[pl.ds(c, M_PER), :]
            if h < N_DEV - 2:
                comm_ref[h + 1] = acc.astype(jnp.bfloat16)
            else:
                rms = jnp.sqrt(jnp.mean(acc * acc, axis=-1, keepdims=True) + 1e-6)
                out_ref[...] = acc / rms * g_ref[...]

    return pl.pallas_call(
        body,
        out_shape=jax.ShapeDtypeStruct((M_PER, D), jnp.float32),
        in_specs=[
            pl.BlockSpec(memory_space=pltpu.VMEM),
            pl.BlockSpec(memory_space=pltpu.VMEM),
        ],
        out_specs=pl.BlockSpec(memory_space=pltpu.VMEM),
        scratch_shapes=[
            pltpu.VMEM((N_DEV, M_PER, D), jnp.bfloat16),
            pltpu.SemaphoreType.DMA((N_DEV - 1,)),
            pltpu.SemaphoreType.DMA((N_DEV - 1,)),
        ],
        compiler_params=pltpu.CompilerParams(collective_id=0),
    )(partial, gamma)


# device time: 53596 ns/iter; 1.5937x vs baseline; 1.5937x over previous
import jax
import jax.numpy as jnp
from jax import lax
from jax.experimental import pallas as pl
from jax.experimental.pallas import tpu as pltpu

NX, NY, NZ = 2, 4, 4
M_TOT, D = 4096, 1024
M_PER = M_TOT // NY
NSUB = NX * NZ
SUB = M_PER // NSUB


def kernel(partial, gamma):
    partial = partial.reshape(M_TOT, D)
    gamma = gamma.reshape(1, D)

    def body(p_ref, g_ref, out_ref,
             ycomm, ysend, yrecv,
             zcomm, zsend, zrecv,
             xbuf, xsend, xrecv):
        x = lax.axis_index("x")
        j = lax.axis_index("y")
        z = lax.axis_index("z")
        y_r = (j + 1) % NY
        y_l = (j - 1) % NY
        z_r = (z + 1) % NZ
        z_l = (z - 1) % NZ
        x_p = 1 - x
        sub_base = (x * NZ + z) * SUB

        barrier_sem = pltpu.get_barrier_semaphore()
        peers = ((x, y_l, z), (x, y_r, z), (x, j, z_l), (x, j, z_r), (x_p, j, z))
        for dev in peers:
            pl.semaphore_signal(
                barrier_sem, inc=1,
                device_id=dev, device_id_type=pl.DeviceIdType.MESH,
            )
        pl.semaphore_wait(barrier_sem, len(peers))

        ycomm[0] = p_ref[
            pl.ds(((j - 1) % NY) * M_PER + sub_base, SUB), :
        ].astype(jnp.bfloat16)
        acc = None
        for h in range(NY - 1):
            rdma = pltpu.make_async_remote_copy(
                src_ref=ycomm.at[h],
                dst_ref=ycomm.at[h + 1],
                send_sem=ysend.at[h],
                recv_sem=yrecv.at[h],
                device_id=(x, y_r, z),
                device_id_type=pl.DeviceIdType.MESH,
            )
            rdma.start()
            rdma.wait()
            c = (j - 2 - h) % NY
            acc = (
                ycomm[h + 1].astype(jnp.float32)
                + p_ref[pl.ds(c * M_PER + sub_base, SUB), :]
            )
            if h < NY - 2:
                ycomm[h + 1] = acc.astype(jnp.bfloat16)

        rms = jnp.sqrt(jnp.mean(acc * acc, axis=-1, keepdims=True) + 1e-6)
        res = acc / rms * g_ref[...]
        out_ref[pl.ds(sub_base, SUB), :] = res
        zcomm[0] = res.astype(jnp.bfloat16)

        for h in range(NZ - 1):
            rdma = pltpu.make_async_remote_copy(
                src_ref=zcomm.at[h],
                dst_ref=zcomm.at[h + 1],
                send_sem=zsend.at[h],
                recv_sem=zrecv.at[h],
                device_id=(x, j, z_r),
                device_id_type=pl.DeviceIdType.MESH,
            )
            rdma.start()
            rdma.wait()

        xr = pltpu.make_async_remote_copy(
            src_ref=zcomm,
            dst_ref=xbuf,
            send_sem=xsend,
            recv_sem=xrecv,
            device_id=(x_p, j, z),
            device_id_type=pl.DeviceIdType.MESH,
        )
        xr.start()
        xr.wait()

        for t in range(NZ):
            zo = (z - t) % NZ
            if t > 0:
                out_ref[pl.ds((x * NZ + zo) * SUB, SUB), :] = (
                    zcomm[t].astype(jnp.float32)
                )
            out_ref[pl.ds((x_p * NZ + zo) * SUB, SUB), :] = (
                xbuf[t].astype(jnp.float32)
            )

    return pl.pallas_call(
        body,
        out_shape=jax.ShapeDtypeStruct((M_PER, D), jnp.float32),
        in_specs=[
            pl.BlockSpec(memory_space=pltpu.VMEM),
            pl.BlockSpec(memory_space=pltpu.VMEM),
        ],
        out_specs=pl.BlockSpec(memory_space=pltpu.VMEM),
        scratch_shapes=[
            pltpu.VMEM((NY, SUB, D), jnp.bfloat16),
            pltpu.SemaphoreType.DMA((NY - 1,)),
            pltpu.SemaphoreType.DMA((NY - 1,)),
            pltpu.VMEM((NZ, SUB, D), jnp.bfloat16),
            pltpu.SemaphoreType.DMA((NZ - 1,)),
            pltpu.SemaphoreType.DMA((NZ - 1,)),
            pltpu.VMEM((NZ, SUB, D), jnp.bfloat16),
            pltpu.SemaphoreType.DMA,
            pltpu.SemaphoreType.DMA,
        ],
        compiler_params=pltpu.CompilerParams(collective_id=0),
    )(partial, gamma)


# device time: 49203 ns/iter; 1.7360x vs baseline; 1.0893x over previous
import jax
import jax.numpy as jnp
from jax import lax
from jax.experimental import pallas as pl
from jax.experimental.pallas import tpu as pltpu

NX, NY, NZ = 2, 4, 4
M_TOT, D = 4096, 1024
M_PER = M_TOT // NY
NSUB = NX * NZ
SUB = M_PER // NSUB


def kernel(partial, gamma):
    sub_base = (lax.axis_index("x") * NZ + lax.axis_index("z")) * SUB
    p4 = lax.dynamic_slice(
        partial.reshape(NY, M_PER, D), (0, sub_base, 0), (NY, SUB, D)
    ).reshape(NY * SUB, D)
    gamma = gamma.reshape(1, D)

    def body(pbuf, g_ref, out_ref,
             ycomm, ysend, yrecv,
             zcomm, zsend, zrecv,
             xbuf, xsend, xrecv):
        x = lax.axis_index("x")
        j = lax.axis_index("y")
        z = lax.axis_index("z")
        y_r = (j + 1) % NY
        z_r = (z + 1) % NZ
        x_p = 1 - x
        sub_base = (x * NZ + z) * SUB

        barrier_sem = pltpu.get_barrier_semaphore()
        peers = (
            (x, (j - 1) % NY, z), (x, y_r, z),
            (x, j, (z - 1) % NZ), (x, j, z_r),
            (x_p, j, z),
        )
        for dev in peers:
            pl.semaphore_signal(
                barrier_sem, inc=1,
                device_id=dev, device_id_type=pl.DeviceIdType.MESH,
            )
        pl.semaphore_wait(barrier_sem, len(peers))

        ycomm[0] = pbuf[pl.ds(((j - 1) % NY) * SUB, SUB), :].astype(jnp.bfloat16)
        acc = None
        for h in range(NY - 1):
            rdma = pltpu.make_async_remote_copy(
                src_ref=ycomm.at[h],
                dst_ref=ycomm.at[h + 1],
                send_sem=ysend.at[h],
                recv_sem=yrecv.at[h],
                device_id=(x, y_r, z),
                device_id_type=pl.DeviceIdType.MESH,
            )
            rdma.start()
            rdma.wait()
            c = (j - 2 - h) % NY
            acc = (
                ycomm[h + 1].astype(jnp.float32)
                + pbuf[pl.ds(c * SUB, SUB), :]
            )
            if h < NY - 2:
                ycomm[h + 1] = acc.astype(jnp.bfloat16)

        rms = jnp.sqrt(jnp.mean(acc * acc, axis=-1, keepdims=True) + 1e-6)
        res = acc / rms * g_ref[...]
        out_ref[pl.ds(sub_base, SUB), :] = res
        zcomm[0] = res.astype(jnp.bfloat16)

        for h in range(NZ - 1):
            rdma = pltpu.make_async_remote_copy(
                src_ref=zcomm.at[h],
                dst_ref=zcomm.at[h + 1],
                send_sem=zsend.at[h],
                recv_sem=zrecv.at[h],
                device_id=(x, j, z_r),
                device_id_type=pl.DeviceIdType.MESH,
            )
            rdma.start()
            rdma.wait()

        xr = pltpu.make_async_remote_copy(
            src_ref=zcomm,
            dst_ref=xbuf,
            send_sem=xsend.at[0],
            recv_sem=xrecv.at[0],
            device_id=(x_p, j, z),
            device_id_type=pl.DeviceIdType.MESH,
        )
        xr.start()
        xr.wait()

        for t in range(NZ):
            zo = (z - t) % NZ
            if t > 0:
                out_ref[pl.ds((x * NZ + zo) * SUB, SUB), :] = (
                    zcomm[t].astype(jnp.float32)
                )
            out_ref[pl.ds((x_p * NZ + zo) * SUB, SUB), :] = (
                xbuf[t].astype(jnp.float32)
            )

    return pl.pallas_call(
        body,
        out_shape=jax.ShapeDtypeStruct((M_PER, D), jnp.float32),
        in_specs=[
            pl.BlockSpec(memory_space=pltpu.VMEM),
            pl.BlockSpec(memory_space=pltpu.VMEM),
        ],
        out_specs=pl.BlockSpec(memory_space=pltpu.VMEM),
        scratch_shapes=[
            pltpu.VMEM((NY, SUB, D), jnp.bfloat16),
            pltpu.SemaphoreType.DMA((NY - 1,)),
            pltpu.SemaphoreType.DMA((NY - 1,)),
            pltpu.VMEM((NZ, SUB, D), jnp.bfloat16),
            pltpu.SemaphoreType.DMA((NZ - 1,)),
            pltpu.SemaphoreType.DMA((NZ - 1,)),
            pltpu.VMEM((NZ, SUB, D), jnp.bfloat16),
            pltpu.SemaphoreType.DMA((NZ,)),
            pltpu.SemaphoreType.DMA((NZ,)),
        ],
        compiler_params=pltpu.CompilerParams(collective_id=0),
    )(p4, gamma)
